# baseline (speedup 1.0000x reference)
import functools

import jax
import jax.numpy as jnp
from jax import lax
from jax.experimental import pallas as pl
from jax.experimental.pallas import tpu as pltpu
from jax.experimental.pallas import tpu_sc as plsc

_NC = 2
_NS = 16
_NW = _NC * _NS


def _sc_embed_gather(embed, ids_pad, rows_per_tile, chunk):
    V, D = embed.shape
    nchunk = rows_per_tile // chunk
    mesh = plsc.VectorSubcoreMesh(core_axis_name="c", subcore_axis_name="s")

    @functools.partial(
        pl.kernel,
        out_type=jax.ShapeDtypeStruct((_NW * rows_per_tile, D), jnp.float32),
        mesh=mesh,
        scratch_types=[
            pltpu.VMEM((nchunk, chunk), jnp.int32),
            pltpu.VMEM((chunk, D), jnp.float32),
            pltpu.SemaphoreType.DMA,
        ],
    )
    def k(emb_hbm, ids_hbm, out_hbm, idx_v, buf_v, sem):
        cid = lax.axis_index("c")
        sid = lax.axis_index("s")
        wid = cid * _NS + sid
        pltpu.sync_copy(ids_hbm.at[wid], idx_v)
        for j in range(nchunk):
            pltpu.async_copy(emb_hbm.at[idx_v.at[j]], buf_v, sem).wait()
            pltpu.sync_copy(
                buf_v, out_hbm.at[pl.ds(wid * rows_per_tile + j * chunk, chunk)])

    return k(embed, ids_pad)


_NBUF = 1
_DBITS = 14


def _sc_edge_aggregate(table, pidx, zeros_rows, np_rows, ch, ck):
    TN, D = table.shape
    ch0, ch1 = ch
    rpt = np_rows // _NS
    mesh = plsc.VectorSubcoreMesh(core_axis_name="c", subcore_axis_name="s")

    @functools.partial(
        pl.kernel,
        out_type=jax.ShapeDtypeStruct((_NC, np_rows, D), jnp.float32),
        mesh=mesh,
        scratch_types=[
            pltpu.VMEM((ch0, ck), jnp.int32),
            pltpu.VMEM((ck,), jnp.int32),
            pltpu.VMEM((ck,), jnp.int32),
            pltpu.VMEM((ck, D), jnp.float32),
            pltpu.VMEM_SHARED((np_rows, D), jnp.float32),
            pltpu.SemaphoreType.DMA,
            pltpu.SemaphoreType.DMA,
        ],
    )
    def k(tab, pi, zr, out, pi_v, cidx, didx, buf, acc, sg, ss):
        cid = lax.axis_index("c")
        sid = lax.axis_index("s")
        wid = cid * _NS + sid
        pltpu.sync_copy(pi.at[wid], pi_v)
        pltpu.sync_copy(zr, acc.at[pl.ds(sid * rpt, rpt)])
        plsc.subcore_barrier()

        def body(j, carry):
            for i in range(ck // 16):
                v = pi_v[j, pl.ds(i * 16, 16)]
                cidx[pl.ds(i * 16, 16)] = lax.shift_right_logical(v, _DBITS)
            pltpu.async_copy(tab.at[cidx], buf, sg)
            for i in range(ck // 16):
                v = pi_v[j, pl.ds(i * 16, 16)]
                didx[pl.ds(i * 16, 16)] = lax.bitwise_and(
                    v, (1 << _DBITS) - 1)
            pltpu.make_async_copy(tab.at[cidx], buf, sg).wait()
            pltpu.async_copy(buf, acc.at[didx], ss, add=True)
            pltpu.make_async_copy(buf, acc.at[didx], ss).wait()
            return carry

        trip = lax.select(cid == 0, ch0, ch1)
        lax.fori_loop(0, trip, body, 0)

        plsc.subcore_barrier()
        pltpu.sync_copy(acc.at[pl.ds(sid * rpt, rpt)],
                        out.at[cid, pl.ds(sid * rpt, rpt)])

    return k(table, pidx, zeros_rows)


def _tc_prep(h, w0, ee, n_rows, bn):
    D = w0.shape[0]
    grid = n_rows // bn

    def body(h_ref, w_ref, ee_ref, out_ref):
        m = jnp.dot(h_ref[...], w_ref[...], preferred_element_type=jnp.float32)
        rm = jnp.mean(ee_ref[...], axis=1)
        out_ref[...] = rm[:, None, None] * m[None, :, :]

    return pl.pallas_call(
        body,
        grid=(grid,),
        in_specs=[
            pl.BlockSpec((bn, D), lambda i: (i, 0)),
            pl.BlockSpec((D, D), lambda i: (0, 0)),
            pl.BlockSpec((7, D), lambda i: (0, 0)),
        ],
        out_specs=pl.BlockSpec((7, bn, D), lambda i: (0, i, 0)),
        out_shape=jax.ShapeDtypeStruct((7, n_rows, D), jnp.float32),
    )(h, w0, ee)


def _gru_math(p_ref, h_ref, wih_ref, whh_ref, bih_ref, bhh_ref):
    D = h_ref.shape[-1]
    agg = p_ref[0] + p_ref[1]
    h = h_ref[...]
    gi = lax.dot_general(agg, wih_ref[...],
                         (((1,), (1,)), ((), ()))) + bih_ref[...]
    gh = lax.dot_general(h, whh_ref[...],
                         (((1,), (1,)), ((), ()))) + bhh_ref[...]
    r = jax.nn.sigmoid(gi[:, :D] + gh[:, :D])
    z = jax.nn.sigmoid(gi[:, D:2 * D] + gh[:, D:2 * D])
    n = jnp.tanh(gi[:, 2 * D:] + r * gh[:, 2 * D:])
    return (1.0 - z) * n + z * h


def _tc_gru_mid(p, h, wih, whh, bih, bhh, wnext, ee, n_rows, bn):
    D = h.shape[-1]
    grid = n_rows // bn
    np_rows = p.shape[1]

    def body(p_ref, h_ref, wih_ref, whh_ref, bih_ref, bhh_ref, wn_ref, ee_ref,
             hout_ref, mout_ref):
        hn = _gru_math(p_ref, h_ref, wih_ref, whh_ref, bih_ref, bhh_ref)
        hout_ref[...] = hn
        m = jnp.dot(hn, wn_ref[...], preferred_element_type=jnp.float32)
        rm = jnp.mean(ee_ref[...], axis=1)
        mout_ref[...] = rm[:, None, None] * m[None, :, :]

    return pl.pallas_call(
        body,
        grid=(grid,),
        in_specs=[
            pl.BlockSpec((2, bn, D), lambda i: (0, i, 0)),
            pl.BlockSpec((bn, D), lambda i: (i, 0)),
            pl.BlockSpec((3 * D, D), lambda i: (0, 0)),
            pl.BlockSpec((3 * D, D), lambda i: (0, 0)),
            pl.BlockSpec((1, 3 * D), lambda i: (0, 0)),
            pl.BlockSpec((1, 3 * D), lambda i: (0, 0)),
            pl.BlockSpec((D, D), lambda i: (0, 0)),
            pl.BlockSpec((7, D), lambda i: (0, 0)),
        ],
        out_specs=[
            pl.BlockSpec((bn, D), lambda i: (i, 0)),
            pl.BlockSpec((7, bn, D), lambda i: (0, i, 0)),
        ],
        out_shape=[
            jax.ShapeDtypeStruct((n_rows, D), jnp.float32),
            jax.ShapeDtypeStruct((7, n_rows, D), jnp.float32),
        ],
    )(p, h, wih, whh, bih, bhh, wnext, ee)


def _tc_gru_last(p, h, wih, whh, bih, bhh, n_rows, bn):
    D = h.shape[-1]
    grid = n_rows // bn

    def body(p_ref, h_ref, wih_ref, whh_ref, bih_ref, bhh_ref, hout_ref):
        hout_ref[...] = _gru_math(p_ref, h_ref, wih_ref, whh_ref, bih_ref,
                                  bhh_ref)

    return pl.pallas_call(
        body,
        grid=(grid,),
        in_specs=[
            pl.BlockSpec((2, bn, D), lambda i: (0, i, 0)),
            pl.BlockSpec((bn, D), lambda i: (i, 0)),
            pl.BlockSpec((3 * D, D), lambda i: (0, 0)),
            pl.BlockSpec((3 * D, D), lambda i: (0, 0)),
            pl.BlockSpec((1, 3 * D), lambda i: (0, 0)),
            pl.BlockSpec((1, 3 * D), lambda i: (0, 0)),
        ],
        out_specs=pl.BlockSpec((bn, D), lambda i: (i, 0)),
        out_shape=jax.ShapeDtypeStruct((n_rows, D), jnp.float32),
    )(p, h, wih, whh, bih, bhh)


def _tc_pool(h, wg, bg, wout, bout):
    n_rows, D = h.shape
    ncls = wout.shape[0]

    def body(bg_ref, h_ref, wg_ref, wout_ref, bout_ref, out_ref):
        h = h_ref[...]
        g = jnp.sum(h * wg_ref[...], axis=1, keepdims=True)
        g = jax.nn.sigmoid(g + bg_ref[0, 0])
        e = jnp.exp(g)
        s = jnp.sum(e)
        num = jnp.sum(e * h, axis=0, keepdims=True)
        hg = num / s
        out_ref[...] = lax.dot_general(
            hg, wout_ref[...], (((1,), (1,)), ((), ()))) + bout_ref[...]

    return pl.pallas_call(
        body,
        grid=(1,),
        in_specs=[
            pl.BlockSpec(memory_space=pltpu.SMEM),
            pl.BlockSpec((n_rows, D), lambda i: (0, 0)),
            pl.BlockSpec((1, D), lambda i: (0, 0)),
            pl.BlockSpec((ncls, D), lambda i: (0, 0)),
            pl.BlockSpec((1, ncls), lambda i: (0, 0)),
        ],
        out_specs=pl.BlockSpec((1, ncls), lambda i: (0, 0)),
        out_shape=jax.ShapeDtypeStruct((1, ncls), jnp.float32),
    )(bg, h, wg, wout, bout)


def kernel(node_ids, edges, edge_types, embed, edge_embed, W, W_ih, W_hh,
           b_ih, b_hh, W_gate, b_gate, W_out, b_out):
    V, D = embed.shape
    N = node_ids.shape[0]
    E = edges.shape[1]
    L = W.shape[0]

    t = ((edge_types.astype(jnp.int32) - 1) % 7)
    src = edges[0].astype(jnp.int32)
    dst = edges[1].astype(jnp.int32)

    ck = 128
    f0 = 0.58
    e0 = int(E * f0)
    ch0 = -(-(-(-e0 // _NS)) // ck)
    cap0 = _NS * ch0 * ck
    e1 = E - cap0
    ch1 = -(-(-(-e1 // _NS)) // ck)
    cap1 = _NS * ch1 * ck
    pad = cap1 - e1
    assert ch1 <= ch0 and cap0 + e1 == E
    pidx_flat = ((t * N + src) << _DBITS) | dst
    p0 = pidx_flat[:cap0].reshape(_NS, ch0, ck)
    p1 = jnp.concatenate(
        [pidx_flat[cap0:], jnp.full((pad,), N, jnp.int32)]
    ).reshape(_NS, ch1, ck)
    p1 = jnp.concatenate(
        [p1, jnp.full((_NS, ch0 - ch1, ck), N, jnp.int32)], axis=1)
    pidx = jnp.concatenate([p0, p1], axis=0)

    np_rows = -(-(N + 1) // (_NS * 8)) * (_NS * 8)
    rpt = np_rows // _NS
    zeros_rows = jnp.zeros((rpt, D), jnp.float32)

    emb_chunk = 64
    rows_per_tile = -(-N // (_NW * emb_chunk)) * emb_chunk
    ids_pad = jnp.concatenate(
        [node_ids.astype(jnp.int32),
         jnp.zeros((_NW * rows_per_tile - N,), jnp.int32)]
    ).reshape(_NW, rows_per_tile // emb_chunk, emb_chunk)

    bih2 = b_ih.reshape(1, -1)
    bhh2 = b_hh.reshape(1, -1)
    bg2 = b_gate.reshape(1, 1)
    bout2 = b_out.reshape(1, -1)

    bn = 2000

    x_pad = _sc_embed_gather(embed, ids_pad, rows_per_tile, emb_chunk)
    h = x_pad[:N]
    table = _tc_prep(h, W[0], edge_embed, N, bn).reshape(7 * N, D)
    for i in range(L):
        partials = _sc_edge_aggregate(table, pidx, zeros_rows,
                                      np_rows, (ch0, ch1), ck)
        if i + 1 < L:
            h, table = _tc_gru_mid(partials, h, W_ih, W_hh, bih2, bhh2,
                                   W[i + 1], edge_embed, N, bn)
            table = table.reshape(7 * N, D)
        else:
            h = _tc_gru_last(partials, h, W_ih, W_hh, bih2, bhh2, N, bn)
    return _tc_pool(h, W_gate, bg2, W_out, bout2)


# speedup vs baseline: 1.0231x; 1.0231x over previous
import functools

import jax
import jax.numpy as jnp
from jax import lax
from jax.experimental import pallas as pl
from jax.experimental.pallas import tpu as pltpu
from jax.experimental.pallas import tpu_sc as plsc

_NC = 2
_NS = 16
_NW = _NC * _NS


def _sc_embed_gather(embed, ids_pad, nc, chunk):
    V, D = embed.shape
    nc0, nc1 = nc
    cap0 = _NS * nc0 * chunk
    n_out = cap0 + _NS * nc1 * chunk
    mesh = plsc.VectorSubcoreMesh(core_axis_name="c", subcore_axis_name="s")

    @functools.partial(
        pl.kernel,
        out_type=jax.ShapeDtypeStruct((n_out, D), jnp.float32),
        mesh=mesh,
        scratch_types=[
            pltpu.VMEM((nc0, chunk), jnp.int32),
            pltpu.VMEM((chunk, D), jnp.float32),
            pltpu.SemaphoreType.DMA,
        ],
    )
    def k(emb_hbm, ids_hbm, out_hbm, idx_v, buf_v, sem):
        cid = lax.axis_index("c")
        sid = lax.axis_index("s")
        wid = cid * _NS + sid
        pltpu.sync_copy(ids_hbm.at[wid], idx_v)
        base = lax.select(cid == 0, wid * nc0 * chunk,
                          cap0 + sid * nc1 * chunk)

        def body(j, carry):
            pltpu.async_copy(emb_hbm.at[idx_v.at[j]], buf_v, sem).wait()
            pltpu.sync_copy(
                buf_v, out_hbm.at[pl.ds(base + j * chunk, chunk)])
            return carry

        lax.fori_loop(0, lax.select(cid == 0, nc0, nc1), body, 0)

    return k(embed, ids_pad)


_NBUF = 1
_DBITS = 14


def _sc_edge_aggregate(table, pidx, zeros_rows, np_rows, ch, ck):
    TN, D = table.shape
    ch0, ch1 = ch
    rpt = np_rows // _NS
    mesh = plsc.VectorSubcoreMesh(core_axis_name="c", subcore_axis_name="s")

    @functools.partial(
        pl.kernel,
        out_type=jax.ShapeDtypeStruct((_NC, np_rows, D), jnp.float32),
        mesh=mesh,
        scratch_types=[
            pltpu.VMEM((ch0, ck), jnp.int32),
            pltpu.VMEM((ck,), jnp.int32),
            pltpu.VMEM((ck,), jnp.int32),
            pltpu.VMEM((ck, D), jnp.float32),
            pltpu.VMEM_SHARED((np_rows, D), jnp.float32),
            pltpu.SemaphoreType.DMA,
            pltpu.SemaphoreType.DMA,
        ],
    )
    def k(tab, pi, zr, out, pi_v, cidx, didx, buf, acc, sg, ss):
        cid = lax.axis_index("c")
        sid = lax.axis_index("s")
        wid = cid * _NS + sid
        pltpu.sync_copy(pi.at[wid], pi_v)
        pltpu.sync_copy(zr, acc.at[pl.ds(sid * rpt, rpt)])
        plsc.subcore_barrier()

        def body(j, carry):
            for i in range(ck // 16):
                v = pi_v[j, pl.ds(i * 16, 16)]
                cidx[pl.ds(i * 16, 16)] = lax.shift_right_logical(v, _DBITS)
            pltpu.async_copy(tab.at[cidx], buf, sg)
            for i in range(ck // 16):
                v = pi_v[j, pl.ds(i * 16, 16)]
                didx[pl.ds(i * 16, 16)] = lax.bitwise_and(
                    v, (1 << _DBITS) - 1)
            pltpu.make_async_copy(tab.at[cidx], buf, sg).wait()
            pltpu.async_copy(buf, acc.at[didx], ss, add=True)
            pltpu.make_async_copy(buf, acc.at[didx], ss).wait()
            return carry

        trip = lax.select(cid == 0, ch0, ch1)
        lax.fori_loop(0, trip, body, 0)

        plsc.subcore_barrier()
        pltpu.sync_copy(acc.at[pl.ds(sid * rpt, rpt)],
                        out.at[cid, pl.ds(sid * rpt, rpt)])

    return k(table, pidx, zeros_rows)


def _tc_prep(h, w0, ee, n_rows, bn):
    D = w0.shape[0]
    grid = n_rows // bn

    def body(h_ref, w_ref, ee_ref, out_ref):
        m = jnp.dot(h_ref[...], w_ref[...], preferred_element_type=jnp.float32)
        rm = jnp.mean(ee_ref[...], axis=1)
        out_ref[...] = rm[:, None, None] * m[None, :, :]

    return pl.pallas_call(
        body,
        grid=(grid,),
        in_specs=[
            pl.BlockSpec((bn, D), lambda i: (i, 0)),
            pl.BlockSpec((D, D), lambda i: (0, 0)),
            pl.BlockSpec((7, D), lambda i: (0, 0)),
        ],
        out_specs=pl.BlockSpec((7, bn, D), lambda i: (0, i, 0)),
        out_shape=jax.ShapeDtypeStruct((7, n_rows, D), jnp.float32),
    )(h, w0, ee)


def _gru_math(p_ref, h_ref, wih_ref, whh_ref, bih_ref, bhh_ref):
    D = h_ref.shape[-1]
    agg = p_ref[0] + p_ref[1]
    h = h_ref[...]
    gi = lax.dot_general(agg, wih_ref[...],
                         (((1,), (1,)), ((), ()))) + bih_ref[...]
    gh = lax.dot_general(h, whh_ref[...],
                         (((1,), (1,)), ((), ()))) + bhh_ref[...]
    r = jax.nn.sigmoid(gi[:, :D] + gh[:, :D])
    z = jax.nn.sigmoid(gi[:, D:2 * D] + gh[:, D:2 * D])
    n = jnp.tanh(gi[:, 2 * D:] + r * gh[:, 2 * D:])
    return (1.0 - z) * n + z * h


def _tc_gru_mid(p, h, wih, whh, bih, bhh, wnext, ee, n_rows, bn):
    D = h.shape[-1]
    grid = n_rows // bn
    np_rows = p.shape[1]

    def body(p_ref, h_ref, wih_ref, whh_ref, bih_ref, bhh_ref, wn_ref, ee_ref,
             hout_ref, mout_ref):
        hn = _gru_math(p_ref, h_ref, wih_ref, whh_ref, bih_ref, bhh_ref)
        hout_ref[...] = hn
        m = jnp.dot(hn, wn_ref[...], preferred_element_type=jnp.float32)
        rm = jnp.mean(ee_ref[...], axis=1)
        mout_ref[...] = rm[:, None, None] * m[None, :, :]

    return pl.pallas_call(
        body,
        grid=(grid,),
        in_specs=[
            pl.BlockSpec((2, bn, D), lambda i: (0, i, 0)),
            pl.BlockSpec((bn, D), lambda i: (i, 0)),
            pl.BlockSpec((3 * D, D), lambda i: (0, 0)),
            pl.BlockSpec((3 * D, D), lambda i: (0, 0)),
            pl.BlockSpec((1, 3 * D), lambda i: (0, 0)),
            pl.BlockSpec((1, 3 * D), lambda i: (0, 0)),
            pl.BlockSpec((D, D), lambda i: (0, 0)),
            pl.BlockSpec((7, D), lambda i: (0, 0)),
        ],
        out_specs=[
            pl.BlockSpec((bn, D), lambda i: (i, 0)),
            pl.BlockSpec((7, bn, D), lambda i: (0, i, 0)),
        ],
        out_shape=[
            jax.ShapeDtypeStruct((n_rows, D), jnp.float32),
            jax.ShapeDtypeStruct((7, n_rows, D), jnp.float32),
        ],
    )(p, h, wih, whh, bih, bhh, wnext, ee)


def _tc_gru_last(p, h, wih, whh, bih, bhh, n_rows, bn):
    D = h.shape[-1]
    grid = n_rows // bn

    def body(p_ref, h_ref, wih_ref, whh_ref, bih_ref, bhh_ref, hout_ref):
        hout_ref[...] = _gru_math(p_ref, h_ref, wih_ref, whh_ref, bih_ref,
                                  bhh_ref)

    return pl.pallas_call(
        body,
        grid=(grid,),
        in_specs=[
            pl.BlockSpec((2, bn, D), lambda i: (0, i, 0)),
            pl.BlockSpec((bn, D), lambda i: (i, 0)),
            pl.BlockSpec((3 * D, D), lambda i: (0, 0)),
            pl.BlockSpec((3 * D, D), lambda i: (0, 0)),
            pl.BlockSpec((1, 3 * D), lambda i: (0, 0)),
            pl.BlockSpec((1, 3 * D), lambda i: (0, 0)),
        ],
        out_specs=pl.BlockSpec((bn, D), lambda i: (i, 0)),
        out_shape=jax.ShapeDtypeStruct((n_rows, D), jnp.float32),
    )(p, h, wih, whh, bih, bhh)


def _tc_pool(h, wg, bg, wout, bout):
    n_rows, D = h.shape
    ncls = wout.shape[0]

    def body(bg_ref, h_ref, wg_ref, wout_ref, bout_ref, out_ref):
        h = h_ref[...]
        g = jnp.sum(h * wg_ref[...], axis=1, keepdims=True)
        g = jax.nn.sigmoid(g + bg_ref[0, 0])
        e = jnp.exp(g)
        s = jnp.sum(e)
        num = jnp.sum(e * h, axis=0, keepdims=True)
        hg = num / s
        out_ref[...] = lax.dot_general(
            hg, wout_ref[...], (((1,), (1,)), ((), ()))) + bout_ref[...]

    return pl.pallas_call(
        body,
        grid=(1,),
        in_specs=[
            pl.BlockSpec(memory_space=pltpu.SMEM),
            pl.BlockSpec((n_rows, D), lambda i: (0, 0)),
            pl.BlockSpec((1, D), lambda i: (0, 0)),
            pl.BlockSpec((ncls, D), lambda i: (0, 0)),
            pl.BlockSpec((1, ncls), lambda i: (0, 0)),
        ],
        out_specs=pl.BlockSpec((1, ncls), lambda i: (0, 0)),
        out_shape=jax.ShapeDtypeStruct((1, ncls), jnp.float32),
    )(bg, h, wg, wout, bout)


def kernel(node_ids, edges, edge_types, embed, edge_embed, W, W_ih, W_hh,
           b_ih, b_hh, W_gate, b_gate, W_out, b_out):
    V, D = embed.shape
    N = node_ids.shape[0]
    E = edges.shape[1]
    L = W.shape[0]

    t = ((edge_types.astype(jnp.int32) - 1) % 7)
    src = edges[0].astype(jnp.int32)
    dst = edges[1].astype(jnp.int32)

    ck = 128
    f0 = 0.60
    e0 = int(E * f0)
    ch0 = -(-(-(-e0 // _NS)) // ck)
    cap0 = _NS * ch0 * ck
    e1 = E - cap0
    ch1 = -(-(-(-e1 // _NS)) // ck)
    cap1 = _NS * ch1 * ck
    pad = cap1 - e1
    assert ch1 <= ch0 and cap0 + e1 == E
    pidx_flat = ((t * N + src) << _DBITS) | dst
    p0 = pidx_flat[:cap0].reshape(_NS, ch0, ck)
    p1 = jnp.concatenate(
        [pidx_flat[cap0:], jnp.full((pad,), N, jnp.int32)]
    ).reshape(_NS, ch1, ck)
    p1 = jnp.concatenate(
        [p1, jnp.full((_NS, ch0 - ch1, ck), N, jnp.int32)], axis=1)
    pidx = jnp.concatenate([p0, p1], axis=0)

    np_rows = -(-(N + 1) // (_NS * 8)) * (_NS * 8)
    rpt = np_rows // _NS
    zeros_rows = jnp.zeros((rpt, D), jnp.float32)

    emb_chunk = 64
    g0 = int(N * f0)
    nc0 = -(-(-(-g0 // _NS)) // emb_chunk)
    gcap0 = _NS * nc0 * emb_chunk
    g1 = N - gcap0
    nc1 = -(-(-(-g1 // _NS)) // emb_chunk)
    gcap1 = _NS * nc1 * emb_chunk
    assert nc1 <= nc0 and gcap0 + g1 == N
    ids32 = node_ids.astype(jnp.int32)
    i0 = ids32[:gcap0].reshape(_NS, nc0, emb_chunk)
    i1 = jnp.concatenate(
        [ids32[gcap0:], jnp.zeros((gcap1 - g1,), jnp.int32)]
    ).reshape(_NS, nc1, emb_chunk)
    i1 = jnp.concatenate(
        [i1, jnp.zeros((_NS, nc0 - nc1, emb_chunk), jnp.int32)], axis=1)
    ids_pad = jnp.concatenate([i0, i1], axis=0)

    bih2 = b_ih.reshape(1, -1)
    bhh2 = b_hh.reshape(1, -1)
    bg2 = b_gate.reshape(1, 1)
    bout2 = b_out.reshape(1, -1)

    bn = 2000

    x_pad = _sc_embed_gather(embed, ids_pad, (nc0, nc1), emb_chunk)
    h = x_pad[:N]
    table = _tc_prep(h, W[0], edge_embed, N, bn).reshape(7 * N, D)
    for i in range(L):
        partials = _sc_edge_aggregate(table, pidx, zeros_rows,
                                      np_rows, (ch0, ch1), ck)
        if i + 1 < L:
            h, table = _tc_gru_mid(partials, h, W_ih, W_hh, bih2, bhh2,
                                   W[i + 1], edge_embed, N, bn)
            table = table.reshape(7 * N, D)
        else:
            h = _tc_gru_last(partials, h, W_ih, W_hh, bih2, bhh2, N, bn)
    return _tc_pool(h, W_gate, bg2, W_out, bout2)
